# Initial kernel scaffold; baseline (speedup 1.0000x reference)
import jax
import jax.numpy as jnp
from jax.experimental import pallas as pl


def kernel(nodes, nodes_target, hist_uv, hist_r, adj, u2e, v2e, r2e, Wh, bh, Ws, bs, W1, b1):
    raise NotImplementedError("write your pallas kernel here")


# speedup vs baseline: 4.1448x; 4.1448x over previous
import functools

import jax
import jax.numpy as jnp
from jax import lax
from jax.experimental import pallas as pl
from jax.experimental.pallas import tpu as pltpu
from jax.experimental.pallas import tpu_sc as plsc

N_USERS = 100000
N_ITEMS = 100000
D = 64
B = 4096
L = 50
LP = 56
NW = 32
ROWS_W = B // NW
CHUNK = 2
NCHUNK = ROWS_W // CHUNK


def _proj_body(u_ref, w1_ref, r_ref, w2_ref, bh_ref, pu_ref, prb_ref):
    pu_ref[...] = jnp.dot(u_ref[...], w1_ref[...],
                          preferred_element_type=jnp.float32)
    prb_ref[...] = jnp.dot(r_ref[...], w2_ref[...],
                           preferred_element_type=jnp.float32) + bh_ref[...]


def _project_tables(u2e, wh1, r2e_pad, wh2, bh_row):
    blk = 4000
    grid = N_USERS // blk
    return pl.pallas_call(
        _proj_body,
        grid=(grid,),
        in_specs=[
            pl.BlockSpec((blk, D), lambda i: (i, 0)),
            pl.BlockSpec((D, D), lambda i: (0, 0)),
            pl.BlockSpec((8, D), lambda i: (0, 0)),
            pl.BlockSpec((D, D), lambda i: (0, 0)),
            pl.BlockSpec((1, D), lambda i: (0, 0)),
        ],
        out_specs=[
            pl.BlockSpec((blk, D), lambda i: (i, 0)),
            pl.BlockSpec((8, D), lambda i: (0, 0)),
        ],
        out_shape=[
            jax.ShapeDtypeStruct((N_USERS, D), jnp.float32),
            jax.ShapeDtypeStruct((8, D), jnp.float32),
        ],
    )(u2e, wh1, r2e_pad, wh2, bh_row)


def _sc_body(pu_hbm, v2e_hbm, prb_hbm, idxh_hbm, idxr_hbm, idxa_hbm, nodes_hbm,
             hist_out_hbm, adj_out_hbm, self_out_hbm,
             idxh_v, idxr_v, idxa_v, idxn_v, prb_v,
             hbuf, abuf, sbuf, hist_o, adj_o,
             semh, sema, sems):
    wid = lax.axis_index("s") * 2 + lax.axis_index("c")
    row0 = wid * ROWS_W

    pltpu.sync_copy(idxh_hbm.at[pl.ds(wid * (ROWS_W // CHUNK), NCHUNK)], idxh_v)
    pltpu.sync_copy(idxa_hbm.at[pl.ds(wid * (ROWS_W // CHUNK), NCHUNK)], idxa_v)
    pltpu.sync_copy(idxr_hbm.at[pl.ds(wid * ROWS_W * LP, ROWS_W * LP)], idxr_v)
    pltpu.sync_copy(nodes_hbm.at[pl.ds(row0, ROWS_W)], idxn_v)
    pltpu.sync_copy(prb_hbm, prb_v)

    pltpu.async_copy(v2e_hbm.at[idxn_v], sbuf, sems).wait()
    pltpu.sync_copy(sbuf, self_out_hbm.at[pl.ds(row0, ROWS_W)])

    inv_l = jnp.float32(1.0 / L)
    zero = jnp.zeros((16,), jnp.float32)

    def do_chunk(c):
        pltpu.async_copy(pu_hbm.at[idxh_v.at[c]], hbuf, semh).wait()
        pltpu.async_copy(v2e_hbm.at[idxa_v.at[c]], abuf, sema).wait()
        for r2 in range(CHUNK):
            row_l = c * CHUNK + r2
            base = row_l * LP
            rvecs = [idxr_v[pl.ds(pl.multiple_of(base + 16 * g, 8), 16)]
                     for g in range(4)]
            hacc = [zero] * 4
            aacc = [zero] * 4
            for l in range(L):
                o = r2 * LP + l
                r64 = rvecs[l // 16][l % 16]
                for j in range(4):
                    g = hbuf[o, pl.ds(16 * j, 16)]
                    p = prb_v[pl.ds(pl.multiple_of(r64 + 16 * j, 8), 16)]
                    hacc[j] = hacc[j] + jnp.maximum(g + p, 0.0)
                    aacc[j] = aacc[j] + abuf[o, pl.ds(16 * j, 16)]
            for j in range(4):
                hist_o[row_l, pl.ds(16 * j, 16)] = hacc[j] * inv_l
                adj_o[row_l, pl.ds(16 * j, 16)] = aacc[j] * inv_l

    def chunk_loop(c, carry):
        do_chunk(c)
        return carry

    lax.fori_loop(0, NCHUNK, chunk_loop, 0)

    pltpu.sync_copy(hist_o, hist_out_hbm.at[pl.ds(row0, ROWS_W)])
    pltpu.sync_copy(adj_o, adj_out_hbm.at[pl.ds(row0, ROWS_W)])


def _sc_gather_agg(pu, v2e, prb_flat, idxh, idxr_flat, idxa, nodes):
    mesh = plsc.VectorSubcoreMesh(core_axis_name="c", subcore_axis_name="s")
    f32 = jnp.float32
    kern = functools.partial(
        pl.kernel,
        mesh=mesh,
        compiler_params=pltpu.CompilerParams(use_tc_tiling_on_sc=False),
        out_type=[
            jax.ShapeDtypeStruct((B, D), f32),
            jax.ShapeDtypeStruct((B, D), f32),
            jax.ShapeDtypeStruct((B, D), f32),
        ],
        scratch_types=[
            pltpu.VMEM((NCHUNK, CHUNK * LP), jnp.int32),
            pltpu.VMEM((ROWS_W * LP,), jnp.int32),
            pltpu.VMEM((NCHUNK, CHUNK * LP), jnp.int32),
            pltpu.VMEM((ROWS_W,), jnp.int32),
            pltpu.VMEM((8 * D,), f32),
            pltpu.VMEM((CHUNK * LP, D), f32),
            pltpu.VMEM((CHUNK * LP, D), f32),
            pltpu.VMEM((ROWS_W, D), f32),
            pltpu.VMEM((ROWS_W, D), f32),
            pltpu.VMEM((ROWS_W, D), f32),
            pltpu.SemaphoreType.DMA,
            pltpu.SemaphoreType.DMA,
            pltpu.SemaphoreType.DMA,
        ],
    )(_sc_body)
    return kern(pu, v2e, prb_flat, idxh, idxr_flat, idxa, nodes)


def _combine_body(self_ref, hist_ref, adj_ref, ws_ref, bs_ref,
                  w1a_ref, w1b_ref, b1_ref, out_ref):
    soc = jnp.maximum(
        jnp.dot(adj_ref[...], ws_ref[...], preferred_element_type=jnp.float32)
        + bs_ref[...], 0.0)
    neigh = 0.5 * (hist_ref[...] + soc)
    out = (jnp.dot(self_ref[...], w1a_ref[...], preferred_element_type=jnp.float32)
           + jnp.dot(neigh, w1b_ref[...], preferred_element_type=jnp.float32)
           + b1_ref[...])
    out_ref[...] = jnp.maximum(out, 0.0)


def _combine(self_feats, hist_agg, adj_mean, Ws, bs_row, w1a, w1b, b1_row):
    return pl.pallas_call(
        _combine_body,
        out_shape=jax.ShapeDtypeStruct((B, D), jnp.float32),
    )(self_feats, hist_agg, adj_mean, Ws, bs_row, w1a, w1b, b1_row)


def kernel(nodes, nodes_target, hist_uv, hist_r, adj, u2e, v2e, r2e,
           Wh, bh, Ws, bs, W1, b1):
    del nodes_target

    wh1 = Wh[:D]
    wh2 = Wh[D:]
    w1a = W1[:D]
    w1b = W1[D:]
    r2e_pad = jnp.concatenate(
        [r2e, jnp.zeros((8 - r2e.shape[0], D), jnp.float32)], axis=0)
    bh_row = bh.reshape(1, D)
    bs_row = bs.reshape(1, D)
    b1_row = b1.reshape(1, D)

    pu, prb = _project_tables(u2e, wh1, r2e_pad, wh2, bh_row)

    def pad_lp(a):
        a = a.astype(jnp.int32)
        return jnp.pad(a, ((0, 0), (0, LP - L)))

    idxh = pad_lp(hist_uv).reshape(B // CHUNK, CHUNK * LP)
    idxa = pad_lp(adj).reshape(B // CHUNK, CHUNK * LP)
    idxr_flat = (pad_lp(hist_r) * D).reshape(-1)
    nodes_i = nodes.astype(jnp.int32)
    prb_flat = prb.reshape(-1)

    hist_agg, adj_mean, self_feats = _sc_gather_agg(
        pu, v2e, prb_flat, idxh, idxr_flat, idxa, nodes_i)

    return _combine(self_feats, hist_agg, adj_mean, Ws, bs_row, w1a, w1b, b1_row)
